# Initial kernel scaffold; baseline (speedup 1.0000x reference)
import jax
import jax.numpy as jnp
from jax.experimental import pallas as pl


def kernel(output, prediction, anchors, targets):
    raise NotImplementedError("write your pallas kernel here")


# speedup vs baseline: 5.9349x; 5.9349x over previous
import functools

import jax
import jax.numpy as jnp
from jax import lax
from jax.experimental import pallas as pl
from jax.experimental.pallas import tpu as pltpu
from jax.experimental.pallas import tpu_sc as plsc

_CONF_THRES = 0.5
_EPS = 1e-7
_SC_LANES = 16
_NC = 2
_NS = 16


def _sc_sparse(out5d, tflat, anchors16, T):
    B, A, G, _, C5 = out5d.shape
    CH = _SC_LANES
    NTILES = T // CH

    mesh = plsc.VectorSubcoreMesh(core_axis_name="c", subcore_axis_name="s")

    @functools.partial(
        pl.kernel,
        out_type=(
            jax.ShapeDtypeStruct((T, 96), jnp.float32),
            jax.ShapeDtypeStruct((T,), jnp.int32),
            jax.ShapeDtypeStruct((T,), jnp.int32),
            jax.ShapeDtypeStruct((4 * T,), jnp.int32),
            jax.ShapeDtypeStruct((4 * T, 16), jnp.float32),
        ),
        mesh=mesh,
        scratch_types=[
            pltpu.VMEM((6 * CH,), jnp.float32),
            pltpu.VMEM((CH,), jnp.float32),
            pltpu.VMEM((CH,), jnp.int32),
            pltpu.VMEM((2, 8, 85), jnp.float32),
            pltpu.VMEM((CH, 96), jnp.float32),
            pltpu.VMEM((4 * CH, 16), jnp.float32),
            pltpu.SemaphoreType.DMA,
        ],
    )
    def k(out_hbm, tT_hbm, anch_hbm, rows_out, f_out, best_out, fck_out,
          o4c_out, t6_v, anch_v, vbi_v, tile_v, rows_v, o4_v, sem):
        wid = lax.axis_index("s") * _NC + lax.axis_index("c")

        @pl.when(wid < NTILES)
        def _():
            base = wid * CH
            for j in range(6):
                pltpu.sync_copy(tT_hbm.at[pl.ds(j * T + base, CH)],
                                t6_v.at[pl.ds(j * CH, CH)])
            pltpu.sync_copy(anch_hbm, anch_v)
            av = anch_v[...]
            g = jnp.float32(G)
            x1 = t6_v[pl.ds(1 * CH, CH)]
            y1 = t6_v[pl.ds(2 * CH, CH)]
            x2 = t6_v[pl.ds(3 * CH, CH)]
            y2 = t6_v[pl.ds(4 * CH, CH)]
            cx = (x1 + x2) * jnp.float32(0.5) * g
            cy = (y1 + y2) * jnp.float32(0.5) * g
            w = (x2 - x1) * g
            h = (y2 - y1) * g
            si = t6_v[pl.ds(0, CH)].astype(jnp.int32)
            gx = cx.astype(jnp.int32)
            gy = cy.astype(jnp.int32)

            ious = []
            for a in range(A):
                aw = av[2 * a]
                ah = av[2 * a + 1]
                inter = jnp.minimum(aw, w) * jnp.minimum(ah, h)
                union = aw * ah + w * h - inter
                ious.append(inter / (jnp.float32(1e-8) + union))
            best = jnp.where(ious[1] > ious[0],
                             jnp.int32(1), jnp.int32(0))
            m01 = jnp.maximum(ious[0], ious[1])
            best = jnp.where(ious[2] > m01, jnp.int32(2), best)

            rowbase = (si * A * G + gy) * G + gx
            f = rowbase + best * (G * G)

            lane = jnp.arange(CH, dtype=jnp.int32)

            vbi_v[...] = f
            pltpu.sync_copy(vbi_v, f_out.at[pl.ds(base, CH)])
            pltpu.sync_copy(vbi_v, fck_out.at[pl.ds(base, CH)])
            vbi_v[...] = best
            pltpu.sync_copy(vbi_v, best_out.at[pl.ds(base, CH)])
            for a in range(A):
                cb = (a + 1) * T + base
                fa = rowbase + a * (G * G)
                valid = ious[a] > jnp.float32(_CONF_THRES)
                vbi_v[...] = jnp.where(valid, fa, -(jnp.int32(cb + 1) + lane))
                pltpu.sync_copy(vbi_v, fck_out.at[pl.ds(cb, CH)])

            NJ = 4 * CH

            def issue(j, slot):
                t = j % CH
                a = j // CH - 1
                bb = si[t]
                yy = gy[t]
                x8 = (gx[t] // 8) * 8
                aa = best[t] if a < 0 else jnp.int32(a)
                return pltpu.async_copy(
                    out_hbm.at[bb, aa, yy, pl.ds(x8, 8)],
                    tile_v.at[slot], sem)

            def process(j, slot):
                t = j % CH
                r = gx[t] - (gx[t] // 8) * 8
                ch0 = tile_v[slot, r, pl.ds(0, 16)]
                if j < CH:
                    for c in range(5):
                        rows_v[t, pl.ds(c * 16, 16)] = \
                            tile_v[slot, r, pl.ds(c * 16, 16)]
                    rows_v[t, pl.ds(69, 16)] = tile_v[slot, r, pl.ds(69, 16)]
                    o4_v[t, :] = ch0
                else:
                    o4_v[j, :] = ch0

            cp = issue(0, 0)
            for j in range(NJ):
                cp_next = issue(j + 1, (j + 1) % 2) if j + 1 < NJ else None
                cp.wait()
                process(j, j % 2)
                cp = cp_next

            pltpu.sync_copy(rows_v, rows_out.at[pl.ds(base, CH), :])
            for cblk in range(4):
                pltpu.sync_copy(
                    o4_v.at[pl.ds(cblk * CH, CH), :],
                    o4c_out.at[pl.ds(cblk * T + base, CH), :])

    return k(out5d, tflat, anchors16)


def _tc_loss(out5d, rows96, tgt, tT, f_r, f_c, best_r, fc_r, fc_c, o4c16,
             anchors6):
    B, A, G, _, C5 = out5d.shape
    C = C5 - 5
    N = B * A * G * G
    T = tgt.shape[0]
    T4 = 4 * T
    g = float(G)

    def body(of_ref, rows_ref, tgt_ref, tT_ref, fr_ref, fc_ref, best_ref,
             fcr_ref, fcc_ref, o4c_ref, anch_ref, out_ref, acc):
        i = pl.program_id(0)

        @pl.when(i == 0)
        def _():
            acc[0] = 0.0

        @pl.when(i < B)
        def _():
            p = jnp.clip(of_ref[0, :, :, :, 4], _EPS, 1.0 - _EPS)
            acc[0] += jnp.sum(jnp.log(1.0 - p))

        @pl.when(i == B)
        def _():
            fr = fr_ref[...]
            fc = fc_ref[0:1, :]
            jio = lax.broadcasted_iota(jnp.int32, (T, T), 1)
            iio = lax.broadcasted_iota(jnp.int32, (T, 1), 0)
            E = fr == fc
            lastj = jnp.max(jnp.where(E, jio, -1), axis=1, keepdims=True)
            winner = (lastj == iio).astype(jnp.float32)
            firstj = jnp.min(jnp.where(E, jio, T), axis=1, keepdims=True)
            firstocc = (firstj == iio).astype(jnp.float32)
            nmask = jnp.sum(firstocc)

            cls_r = tgt_ref[:, 5:6].astype(jnp.int32)
            cls_c = tT_ref[5:6, :].astype(jnp.int32)
            kr = fr * C + cls_r
            kc = fc * C + cls_c
            Ek = kr == kc
            firstk = (jnp.min(jnp.where(Ek, jio, T), axis=1, keepdims=True)
                      == iio).astype(jnp.float32)

            x1 = tgt_ref[:, 1:2]
            y1 = tgt_ref[:, 2:3]
            x2 = tgt_ref[:, 3:4]
            y2 = tgt_ref[:, 4:5]
            cx = (x1 + x2) * 0.5 * g
            cy = (y1 + y2) * 0.5 * g
            w = (x2 - x1) * g
            h = (y2 - y1) * g
            fx = cx - jnp.floor(cx)
            fy = cy - jnp.floor(cy)
            best = best_ref[...]
            aw = jnp.zeros((T, 1), jnp.float32)
            ah = jnp.zeros((T, 1), jnp.float32)
            for a in range(A):
                aw = jnp.where(best == a, anch_ref[0, 2 * a], aw)
                ah = jnp.where(best == a, anch_ref[0, 2 * a + 1], ah)
            tw = jnp.log(1e-8 + w / aw)
            th = jnp.log(1e-8 + h / ah)

            rows = rows_ref[...]
            s_x = jnp.sum(winner * (rows[:, 0:1] - fx) ** 2)
            s_y = jnp.sum(winner * (rows[:, 1:2] - fy) ** 2)
            s_w = jnp.sum(winner * (rows[:, 2:3] - tw) ** 2)
            s_h = jnp.sum(winner * (rows[:, 3:4] - th) ** 2)

            p4 = jnp.clip(rows[:, 4:5], _EPS, 1.0 - _EPS)
            s_obj = jnp.sum(firstocc * (-jnp.log(p4)))

            pcls = jnp.clip(rows[:, 5:5 + C], _EPS, 1.0 - _EPS)
            s_cls_neg = jnp.sum(firstocc * (-jnp.log(1.0 - pcls)))
            onehot = (lax.broadcasted_iota(jnp.int32, (T, C), 1) == cls_r)
            pk = jnp.sum(jnp.where(onehot, pcls, 0.0), axis=1, keepdims=True)
            s_cls_pos = jnp.sum(
                firstk * (-jnp.log(pk) + jnp.log(1.0 - pk)))

            fcr = fcr_ref[...]
            fcc = fcc_ref[0:1, :]
            jio4 = lax.broadcasted_iota(jnp.int32, (T4, T4), 1)
            iio4 = lax.broadcasted_iota(jnp.int32, (T4, 1), 0)
            Ec = fcr == fcc
            firstc = (jnp.min(jnp.where(Ec, jio4, T4), axis=1, keepdims=True)
                      == iio4)
            cnt = jnp.logical_and(firstc, fcr >= 0).astype(jnp.float32)
            ncleared = jnp.sum(cnt)
            pc = jnp.clip(o4c_ref[:, 4:5], _EPS, 1.0 - _EPS)
            s_cl = jnp.sum(cnt * (-jnp.log(1.0 - pc)))

            s_all = -acc[0]
            denom = jnp.maximum(nmask, 1.0)
            m_x = s_x / denom
            m_y = s_y / denom
            m_w = s_w / denom
            m_h = s_h / denom
            obj_loss = s_obj / denom
            noobj_loss = (s_all - s_cl) / jnp.maximum(
                jnp.float32(N) - ncleared, 1.0)
            m_obj = obj_loss + 100.0 * noobj_loss
            m_cls = (s_cls_neg + s_cls_pos) / jnp.maximum(
                nmask * jnp.float32(C), 1.0)
            total = m_x + m_y + m_w + m_h + m_obj + m_cls
            out_ref[...] = total.reshape(1, 1)

    const = lambda shape: pl.BlockSpec(shape, lambda i: tuple(0 for _ in shape))
    return pl.pallas_call(
        body,
        grid=(B + 1,),
        in_specs=[
            pl.BlockSpec((1, A, G, G, C5),
                         lambda i: (jnp.minimum(i, B - 1), 0, 0, 0, 0)),
            const((T, 96)),
            const((T, 8)),
            const((8, T)),
            const((T, 1)),
            const((8, T)),
            const((T, 1)),
            const((T4, 1)),
            const((8, T4)),
            const((T4, 16)),
            pl.BlockSpec(memory_space=pltpu.SMEM),
        ],
        out_specs=pl.BlockSpec((1, 1), lambda i: (0, 0)),
        out_shape=jax.ShapeDtypeStruct((1, 1), jnp.float32),
        scratch_shapes=[pltpu.SMEM((1,), jnp.float32)],
    )(out5d, rows96, tgt, tT, f_r, f_c, best_r, fc_r, fc_c, o4c16, anchors6)


def kernel(output, prediction, anchors, targets):
    B, A, G, _, C5 = prediction.shape
    T = targets.shape[0]

    tT8 = jnp.concatenate([targets.T, jnp.zeros((2, T), jnp.float32)])
    tgt8 = jnp.concatenate([targets, jnp.zeros((T, 2), jnp.float32)], axis=1)
    anchors6 = anchors.reshape(1, 2 * A)
    anchors16 = jnp.concatenate(
        [anchors.reshape(2 * A), jnp.zeros((_SC_LANES - 2 * A,), jnp.float32)])

    rows96, f, best, fckey, o4c16 = _sc_sparse(
        output, targets.T.reshape(6 * T), anchors16, T)

    fc8 = jnp.broadcast_to(f.reshape(1, T), (8, T))
    fcc8 = jnp.broadcast_to(fckey.reshape(1, 4 * T), (8, 4 * T))
    loss = _tc_loss(
        output, rows96, tgt8, tT8,
        f.reshape(T, 1), fc8, best.reshape(T, 1),
        fckey.reshape(4 * T, 1), fcc8, o4c16, anchors6)
    return loss.reshape(())
